# Initial kernel scaffold; baseline (speedup 1.0000x reference)
import jax
import jax.numpy as jnp
from jax.experimental import pallas as pl


def kernel(embed_states, batch_index, W_dev, b_dev, W_act, b_act):
    raise NotImplementedError("write your pallas kernel here")


# speedup vs baseline: 1.3276x; 1.3276x over previous
import functools

import jax
import jax.numpy as jnp
from jax import lax
from jax.experimental import pallas as pl
from jax.experimental.pallas import tpu as pltpu
from jax.experimental.pallas import tpu_sc as plsc

NUM_SC_CORES = 2
NUM_SUBCORES = 16
NUM_WORKERS = NUM_SC_CORES * NUM_SUBCORES
LANES = 16


def _dense_body(x_ref, wd_ref, bd_ref, wa_ref, ba_ref, la_ref, d_ref):
    x = x_ref[...]
    a = jnp.dot(x, wa_ref[...], preferred_element_type=jnp.float32) + ba_ref[...]
    amax = jnp.max(a, axis=-1, keepdims=True)
    lse = jnp.log(jnp.sum(jnp.exp(a - amax), axis=-1, keepdims=True)) + amax
    la_ref[...] = a - lse
    d_ref[...] = (
        jnp.dot(x, wd_ref[...], preferred_element_type=jnp.float32) + bd_ref[...]
    )


def _dense_call(x, w_dev, b_dev, w_act, b_act, block_rows):
    n, e = x.shape
    a = w_act.shape[1]
    grid = (n // block_rows,)
    return pl.pallas_call(
        _dense_body,
        grid=grid,
        in_specs=[
            pl.BlockSpec((block_rows, e), lambda i: (i, 0)),
            pl.BlockSpec((e, 1), lambda i: (0, 0)),
            pl.BlockSpec((1, 1), lambda i: (0, 0)),
            pl.BlockSpec((e, a), lambda i: (0, 0)),
            pl.BlockSpec((1, a), lambda i: (0, 0)),
        ],
        out_specs=[
            pl.BlockSpec((block_rows, a), lambda i: (i, 0)),
            pl.BlockSpec((block_rows, 1), lambda i: (i, 0)),
        ],
        out_shape=[
            jax.ShapeDtypeStruct((n, a), jnp.float32),
            jax.ShapeDtypeStruct((n, 1), jnp.float32),
        ],
    )(x, w_dev, b_dev.reshape(1, 1), w_act, b_act.reshape(1, a))


def _seg_body(num_segments, chunk, d_hbm, idx_hbm, pmax_hbm, psum_hbm,
              dv, iv, lm_ref, s_ref):
    minf = jnp.float32(-jnp.inf)
    wid = lax.axis_index("s") * NUM_SC_CORES + lax.axis_index("c")
    base = wid * chunk
    pltpu.sync_copy(d_hbm.at[pl.ds(base, chunk)], dv)
    pltpu.sync_copy(idx_hbm.at[pl.ds(base, chunk)], iv)

    s_ref[...] = jnp.zeros((LANES,), jnp.float32)

    first = iv[pl.ds(0, LANES)][0]
    last = iv[pl.ds(chunk - LANES, LANES)][LANES - 1]
    nvec = chunk // LANES
    lane = lax.iota(jnp.int32, LANES)

    @pl.when(first == last)
    def _chunk_uniform():
        def body(i, acc):
            return jnp.maximum(acc, dv[pl.ds(i * LANES, LANES)])

        acc = lax.fori_loop(0, nvec, body, jnp.full((LANES,), minf, jnp.float32))
        lm_ref[...] = jnp.where(lane == first, jnp.max(acc), minf)

    @pl.when(first != last)
    def _chunk_mixed():
        def body(i, lm):
            v = dv[pl.ds(i * LANES, LANES)]
            sg = iv[pl.ds(i * LANES, LANES)]
            s0 = sg[0]
            s15 = sg[LANES - 1]

            def vec_uniform(lm):
                return jnp.where(lane == s0, jnp.maximum(lm, jnp.max(v)), lm)

            def vec_mixed(lm):
                def seg_loop(b, lm):
                    mb = jnp.max(jnp.where(sg == b, v, minf))
                    return jnp.where(lane == b, jnp.maximum(lm, mb), lm)

                return lax.fori_loop(0, num_segments, seg_loop, lm)

            return lax.cond(s0 == s15, vec_uniform, vec_mixed, lm)

        lm_ref[...] = lax.fori_loop(
            0, nvec, body, jnp.full((LANES,), minf, jnp.float32))

    def body2(i, carry):
        v = dv[pl.ds(i * LANES, LANES)]
        sg = iv[pl.ds(i * LANES, LANES)]
        shift = plsc.load_gather(lm_ref, [sg])
        plsc.addupdate_scatter(s_ref, [sg], jnp.exp(v - shift))
        return carry

    lax.fori_loop(0, nvec, body2, 0)

    pltpu.sync_copy(lm_ref, pmax_hbm.at[wid])
    pltpu.sync_copy(s_ref, psum_hbm.at[wid])


def _seg_call(d_flat, idx, num_segments):
    n = d_flat.shape[0]
    chunk = n // NUM_WORKERS
    mesh = plsc.VectorSubcoreMesh(
        core_axis_name="c", subcore_axis_name="s",
        num_cores=NUM_SC_CORES, num_subcores=NUM_SUBCORES,
    )
    return pl.kernel(
        functools.partial(_seg_body, num_segments, chunk),
        out_type=[
            jax.ShapeDtypeStruct((NUM_WORKERS, LANES), jnp.float32),
            jax.ShapeDtypeStruct((NUM_WORKERS, LANES), jnp.float32),
        ],
        mesh=mesh,
        compiler_params=pltpu.CompilerParams(needs_layout_passes=False),
        scratch_types=[
            pltpu.VMEM((chunk,), jnp.float32),
            pltpu.VMEM((chunk,), jnp.int32),
            pltpu.VMEM((LANES,), jnp.float32),
            pltpu.VMEM((LANES,), jnp.float32),
        ],
    )(d_flat, idx)


def _combine_body(num_segments, la_ref, d_ref, idx_ref, pmax_ref, psum_ref,
                  out_ref):
    pmax = pmax_ref[...]
    psum = psum_ref[...]
    m = jnp.max(pmax, axis=0)
    s = jnp.sum(psum * jnp.exp(pmax - m[None, :]), axis=0)
    c = m + jnp.log(s)
    onehot = idx_ref[...] == lax.broadcasted_iota(jnp.int32, (1, num_segments), 1)
    crow = jnp.sum(jnp.where(onehot, c[None, :], 0.0), axis=1, keepdims=True)
    out_ref[...] = la_ref[...] + (d_ref[...] - crow)


def _combine_call(la, d, idx2d, pmax, psum, num_segments, block_rows):
    n, a = la.shape
    grid = (n // block_rows,)
    return pl.pallas_call(
        functools.partial(_combine_body, num_segments),
        grid=grid,
        in_specs=[
            pl.BlockSpec((block_rows, a), lambda i: (i, 0)),
            pl.BlockSpec((block_rows, 1), lambda i: (i, 0)),
            pl.BlockSpec((block_rows, 1), lambda i: (i, 0)),
            pl.BlockSpec((NUM_WORKERS, num_segments), lambda i: (0, 0)),
            pl.BlockSpec((NUM_WORKERS, num_segments), lambda i: (0, 0)),
        ],
        out_specs=pl.BlockSpec((block_rows, a), lambda i: (i, 0)),
        out_shape=jax.ShapeDtypeStruct((n, a), jnp.float32),
    )(la, d, idx2d, pmax, psum)


def kernel(embed_states, batch_index, W_dev, b_dev, W_act, b_act):
    n = embed_states.shape[0]
    num_segments = 16
    idx = batch_index.astype(jnp.int32)
    la, d = _dense_call(embed_states, W_dev, b_dev, W_act, b_act,
                        block_rows=2048)
    pmax, psum = _seg_call(d.reshape(n), idx, num_segments)
    return _combine_call(la, d, idx.reshape(n, 1), pmax, psum,
                         num_segments, block_rows=4096)
